# baseline (speedup 1.0000x reference)
import jax
import jax.numpy as jnp
from jax import lax
from jax.experimental import pallas as pl
from jax.experimental.pallas import tpu as pltpu
from jax.experimental.pallas import tpu_sc as plsc

F32 = jnp.float32
I32 = jnp.int32

N = 10000
E = 320000
D = 128
NPAD = 10240
CH = 80
NW = 32
EPT = E // NW
CPT = EPT // CH
NPAIR = CPT // 2
ROWS_PER_TILE = NPAD // 16
_mesh = plsc.VectorSubcoreMesh(core_axis_name="c", subcore_axis_name="s")
_sc_params = pltpu.CompilerParams(needs_layout_passes=False)


def _ids():
    c = lax.axis_index("c")
    s = lax.axis_index("s")
    return c, s, s * 2 + c


_DEG_OUT = jax.ShapeDtypeStruct((2 * NW, NPAD), F32)
_DEG_SCRATCH = [
    pltpu.VMEM((EPT,), I32),
    pltpu.VMEM((EPT,), I32),
    pltpu.VMEM((NPAD,), F32),
    pltpu.VMEM((NPAD,), F32),
]


def _sc_degrees_body(src_hbm, dst_hbm, out_hbm, idx_s, idx_d, acc_o, acc_i):
    c, s, wid = _ids()
    zero = jnp.zeros((16,), F32)
    ones = jnp.ones((16,), F32)

    def zbody(i, carry):
        acc_o[pl.ds(i * 16, 16)] = zero
        acc_i[pl.ds(i * 16, 16)] = zero
        return carry

    lax.fori_loop(0, NPAD // 16, zbody, 0)

    ebase = wid * EPT
    pltpu.sync_copy(src_hbm.at[pl.ds(ebase, EPT)], idx_s)
    pltpu.sync_copy(dst_hbm.at[pl.ds(ebase, EPT)], idx_d)

    def ebody(i, carry):
        iv_s = idx_s[pl.ds(i * 16, 16)]
        plsc.addupdate_scatter(acc_o, [iv_s], ones)
        iv_d = idx_d[pl.ds(i * 16, 16)]
        plsc.addupdate_scatter(acc_i, [iv_d], ones)
        return carry

    lax.fori_loop(0, EPT // 16, ebody, 0)

    pltpu.sync_copy(acc_o, out_hbm.at[wid])
    pltpu.sync_copy(acc_i, out_hbm.at[NW + wid])


NTRIP = CPT // 3

_AGG_OUT = jax.ShapeDtypeStruct((2, NPAD, D), F32)
_AGG_SCRATCH = [
    pltpu.VMEM((CH,), I32), pltpu.VMEM((CH,), I32), pltpu.VMEM((CH,), I32),
    pltpu.VMEM((CH,), I32), pltpu.VMEM((CH,), I32), pltpu.VMEM((CH,), I32),
    pltpu.VMEM((CH, D), F32),
    pltpu.VMEM((CH, D), F32),
    pltpu.VMEM((CH, D), F32),
    pltpu.VMEM_SHARED((NPAD, D), F32),
    pltpu.SemaphoreType.DMA, pltpu.SemaphoreType.DMA,
    pltpu.SemaphoreType.DMA, pltpu.SemaphoreType.DMA,
    pltpu.SemaphoreType.DMA, pltpu.SemaphoreType.DMA,
]


def _sc_aggregate_body(h_hbm, src_hbm, dst_hbm, out_hbm,
                       is0, is1, is2, id0, id1, id2, r0, r1, r2, acc,
                       si0, si1, si2, sg0, sg1, sg2):
    c, s, wid = _ids()
    zero = jnp.zeros((16,), F32)

    def zbody(r, carry):
        for j in range(D // 16):
            r0[r, pl.ds(j * 16, 16)] = zero
        return carry

    lax.fori_loop(0, CH, zbody, 0)
    for t in range(ROWS_PER_TILE // CH):
        pltpu.sync_copy(r0, acc.at[pl.ds(s * ROWS_PER_TILE + t * CH, CH), :])
    plsc.subcore_barrier()

    ebase = wid * EPT

    def idx_fetch(chunk, bi_s, bi_d, sem):
        off = ebase + chunk * CH
        pltpu.async_copy(src_hbm.at[pl.ds(off, CH)], bi_s, sem)
        pltpu.async_copy(dst_hbm.at[pl.ds(off, CH)], bi_d, sem)

    def idx_wait(chunk, bi_s, bi_d, sem):
        off = ebase + chunk * CH
        pltpu.make_async_copy(src_hbm.at[pl.ds(off, CH)], bi_s, sem).wait()
        pltpu.make_async_copy(dst_hbm.at[pl.ds(off, CH)], bi_d, sem).wait()

    def gather(bi_s, buf, sem):
        pltpu.async_copy(h_hbm.at[bi_s], buf, sem)

    def gwait(bi_s, buf, sem):
        pltpu.make_async_copy(h_hbm.at[bi_s], buf, sem).wait()

    idx_fetch(0, is0, id0, si0)
    idx_fetch(1, is1, id1, si1)
    idx_fetch(2, is2, id2, si2)
    idx_wait(0, is0, id0, si0)
    gather(is0, r0, sg0)
    idx_wait(1, is1, id1, si1)
    gather(is1, r1, sg1)

    def trip(kk, carry):
        c0 = 3 * kk
        idx_wait(c0 + 2, is2, id2, si2)
        gather(is2, r2, sg2)
        gwait(is0, r0, sg0)
        pltpu.sync_copy(r0, acc.at[id0], add=True)
        idx_fetch(c0 + 3, is0, id0, si0)
        gwait(is1, r1, sg1)
        pltpu.sync_copy(r1, acc.at[id1], add=True)
        idx_fetch(c0 + 4, is1, id1, si1)
        idx_wait(c0 + 3, is0, id0, si0)
        gather(is0, r0, sg0)
        gwait(is2, r2, sg2)
        pltpu.sync_copy(r2, acc.at[id2], add=True)

        @pl.when(kk < NTRIP - 1)
        def _():
            idx_fetch(c0 + 5, is2, id2, si2)

        idx_wait(c0 + 4, is1, id1, si1)
        gather(is1, r1, sg1)
        return carry

    lax.fori_loop(0, NTRIP, trip, 0)

    gwait(is0, r0, sg0)
    pltpu.sync_copy(r0, acc.at[id0], add=True)
    gwait(is1, r1, sg1)
    pltpu.sync_copy(r1, acc.at[id1], add=True)

    plsc.subcore_barrier()
    for t in range(ROWS_PER_TILE // CH):
        off = s * ROWS_PER_TILE + t * CH
        pltpu.sync_copy(acc.at[pl.ds(off, CH), :],
                        out_hbm.at[c, pl.ds(off, CH), :])


_SCORE_OUT = jax.ShapeDtypeStruct((E,), F32)
_SCORE_SCRATCH = [
    pltpu.VMEM((EPT,), I32),
    pltpu.VMEM((EPT,), I32),
    pltpu.VMEM((CH, D), F32), pltpu.VMEM((CH, D), F32),
    pltpu.VMEM((CH, D), F32), pltpu.VMEM((CH, D), F32),
    pltpu.VMEM((CH, D), F32), pltpu.VMEM((CH, D), F32),
    pltpu.VMEM((512,), F32),
    pltpu.VMEM((EPT,), F32),
    pltpu.SemaphoreType.DMA, pltpu.SemaphoreType.DMA,
    pltpu.SemaphoreType.DMA, pltpu.SemaphoreType.DMA,
    pltpu.SemaphoreType.DMA, pltpu.SemaphoreType.DMA,
]


def _sc_scores_body(x_hbm, src_hbm, dst_hbm, out_hbm, idx_s, idx_d,
                    s0, s1, s2, d0, d1, d2, pad, oall,
                    sa0, sa1, sa2, sb0, sb1, sb2):
    c, s, wid = _ids()
    lanes = lax.iota(I32, 16)

    ebase = wid * EPT
    pltpu.sync_copy(src_hbm.at[pl.ds(ebase, EPT)], idx_s)
    pltpu.sync_copy(dst_hbm.at[pl.ds(ebase, EPT)], idx_d)

    def gathers(chunk, bs, bd, sema, semb):
        pltpu.async_copy(x_hbm.at[idx_s.at[pl.ds(chunk * CH, CH)]], bs, sema)
        pltpu.async_copy(x_hbm.at[idx_d.at[pl.ds(chunk * CH, CH)]], bd, semb)

    def gwait(chunk, bs, bd, sema, semb):
        pltpu.make_async_copy(
            x_hbm.at[idx_s.at[pl.ds(chunk * CH, CH)]], bs, sema).wait()
        pltpu.make_async_copy(
            x_hbm.at[idx_d.at[pl.ds(chunk * CH, CH)]], bd, semb).wait()

    def compute(chunk, bs, bd):
        def scat16(base, poff):
            for e in range(16):
                eidx = base + e
                acc = bs[eidx, pl.ds(0, 16)] * bd[eidx, pl.ds(0, 16)]
                for j in range(1, D // 16):
                    acc = acc + (bs[eidx, pl.ds(j * 16, 16)]
                                 * bd[eidx, pl.ds(j * 16, 16)])
                plsc.store_scatter(pad, [lanes * 16 + (poff + e)], acc)

        def red16(base, poff):
            dots = pad[pl.ds(poff, 16)]
            for r in range(1, 16):
                dots = dots + pad[pl.ds(r * 16 + poff, 16)]
            sig = 1.0 / (1.0 + jnp.exp(-dots))
            oall[pl.ds(chunk * CH + base, 16)] = sig

        def grp2(i, carry):
            b0 = i * 32
            scat16(b0, 0)
            scat16(b0 + 16, 256)
            red16(b0, 0)
            red16(b0 + 16, 256)
            return carry

        lax.fori_loop(0, 2, grp2, 0)
        scat16(64, 0)
        red16(64, 0)

    gathers(0, s0, d0, sa0, sb0)
    gathers(1, s1, d1, sa1, sb1)

    def trip(kk, carry):
        c0 = 3 * kk
        gathers(c0 + 2, s2, d2, sa2, sb2)
        gwait(c0, s0, d0, sa0, sb0)
        compute(c0, s0, d0)
        gathers(c0 + 3, s0, d0, sa0, sb0)
        gwait(c0 + 1, s1, d1, sa1, sb1)
        compute(c0 + 1, s1, d1)
        gathers(c0 + 4, s1, d1, sa1, sb1)
        gwait(c0 + 2, s2, d2, sa2, sb2)
        compute(c0 + 2, s2, d2)
        return carry

    lax.fori_loop(0, NTRIP, trip, 0)

    last = CPT - 2
    gwait(last, s0, d0, sa0, sb0)
    compute(last, s0, d0)
    gwait(last + 1, s1, d1, sa1, sb1)
    compute(last + 1, s1, d1)

    pltpu.sync_copy(oall, out_hbm.at[pl.ds(ebase, EPT)])


def _make_sc(body, out_type, scratch):
    return pl.kernel(
        body,
        out_type=out_type,
        mesh=_mesh,
        compiler_params=_sc_params,
        scratch_types=scratch,
    )


_sc_degrees = _make_sc(_sc_degrees_body, _DEG_OUT, _DEG_SCRATCH)
_sc_aggregate = _make_sc(_sc_aggregate_body, _AGG_OUT, _AGG_SCRATCH)
_sc_scores = _make_sc(_sc_scores_body, _SCORE_OUT, _SCORE_SCRATCH)


_BLK = 1024
_GRID = NPAD // _BLK


def _scales(degp):
    dego = jnp.sum(degp[0:NW], axis=0)[:, None]
    degi = jnp.sum(degp[NW:2 * NW], axis=0)[:, None]
    s_out = lax.rsqrt(jnp.maximum(dego, 1.0))
    s_in = lax.rsqrt(jnp.maximum(degi, 1.0))
    return s_out, s_in


def _tc_prep_body(degp_ref, feat_ref, w1_ref, out_ref):
    s_out, _ = _scales(degp_ref[...])
    out_ref[...] = jnp.dot(feat_ref[...] * s_out, w1_ref[...],
                           preferred_element_type=F32)


def _tc_mid_body(degp_ref, aggp_ref, b1_ref, w2_ref, out_ref):
    s_out, s_in = _scales(degp_ref[...])
    x = aggp_ref[0] + aggp_ref[1]
    x1 = jnp.maximum(x * s_in + b1_ref[...], 0.0)
    out_ref[...] = jnp.dot(x1 * s_out, w2_ref[...],
                           preferred_element_type=F32)


def _tc_fin_body(degp_ref, aggp_ref, b2_ref, out_ref):
    _, s_in = _scales(degp_ref[...])
    x = aggp_ref[0] + aggp_ref[1]
    out_ref[...] = jnp.maximum(x * s_in + b2_ref[...], 0.0)


_deg_spec = pl.BlockSpec((2 * NW, _BLK), lambda i: (0, i))
_row_spec = pl.BlockSpec((_BLK, D), lambda i: (i, 0))
_agg_spec = pl.BlockSpec((2, _BLK, D), lambda i: (0, i, 0))
_w_spec = pl.BlockSpec((D, D), lambda i: (0, 0))
_b_spec = pl.BlockSpec((1, D), lambda i: (0, 0))

_tc_prep = pl.pallas_call(
    _tc_prep_body,
    grid=(_GRID,),
    in_specs=[_deg_spec, _row_spec, _w_spec],
    out_specs=_row_spec,
    out_shape=jax.ShapeDtypeStruct((NPAD, D), F32),
)

_tc_mid = pl.pallas_call(
    _tc_mid_body,
    grid=(_GRID,),
    in_specs=[_deg_spec, _agg_spec, _b_spec, _w_spec],
    out_specs=_row_spec,
    out_shape=jax.ShapeDtypeStruct((NPAD, D), F32),
)

_tc_fin = pl.pallas_call(
    _tc_fin_body,
    grid=(_GRID,),
    in_specs=[_deg_spec, _agg_spec, _b_spec],
    out_specs=_row_spec,
    out_shape=jax.ShapeDtypeStruct((NPAD, D), F32),
)


def kernel(features, edge_index, edge_type, W1, b1, W2, b2):
    del edge_type
    src = edge_index[0]
    dst = edge_index[1]
    featp = jnp.pad(features, ((0, NPAD - N), (0, 0)))

    degp = _sc_degrees(src, dst)
    h1 = _tc_prep(degp, featp, W1)
    agg1 = _sc_aggregate(h1, src, dst)
    h2 = _tc_mid(degp, agg1, b1.reshape(1, D), W2)
    agg2 = _sc_aggregate(h2, src, dst)
    x2 = _tc_fin(degp, agg2, b2.reshape(1, D))
    scores = _sc_scores(x2, src, dst)
    return scores


# speedup vs baseline: 1.3145x; 1.3145x over previous
import jax
import jax.numpy as jnp
from jax import lax
from jax.experimental import pallas as pl
from jax.experimental.pallas import tpu as pltpu
from jax.experimental.pallas import tpu_sc as plsc

F32 = jnp.float32
I32 = jnp.int32

N = 10000
E = 320000
D = 128
NPAD = 10240
CH = 80
NW = 32
EPT = E // NW
CPT = EPT // CH
NPAIR = CPT // 2
ROWS_PER_TILE = NPAD // 16
_mesh = plsc.VectorSubcoreMesh(core_axis_name="c", subcore_axis_name="s")
_sc_params = pltpu.CompilerParams(needs_layout_passes=False)


def _ids():
    c = lax.axis_index("c")
    s = lax.axis_index("s")
    return c, s, s * 2 + c


_DEG_OUT = jax.ShapeDtypeStruct((2 * NW, NPAD), F32)
_DEG_SCRATCH = [
    pltpu.VMEM((EPT,), I32),
    pltpu.VMEM((EPT,), I32),
    pltpu.VMEM((NPAD,), F32),
    pltpu.VMEM((NPAD,), F32),
]


def _sc_degrees_body(src_hbm, dst_hbm, out_hbm, idx_s, idx_d, acc_o, acc_i):
    c, s, wid = _ids()
    zero = jnp.zeros((16,), F32)
    ones = jnp.ones((16,), F32)

    def zbody(i, carry):
        acc_o[pl.ds(i * 16, 16)] = zero
        acc_i[pl.ds(i * 16, 16)] = zero
        return carry

    lax.fori_loop(0, NPAD // 16, zbody, 0)

    ebase = wid * EPT
    pltpu.sync_copy(src_hbm.at[pl.ds(ebase, EPT)], idx_s)
    pltpu.sync_copy(dst_hbm.at[pl.ds(ebase, EPT)], idx_d)

    def ebody(i, carry):
        iv_s = idx_s[pl.ds(i * 16, 16)]
        plsc.addupdate_scatter(acc_o, [iv_s], ones)
        iv_d = idx_d[pl.ds(i * 16, 16)]
        plsc.addupdate_scatter(acc_i, [iv_d], ones)
        return carry

    lax.fori_loop(0, EPT // 16, ebody, 0)

    pltpu.sync_copy(acc_o, out_hbm.at[wid])
    pltpu.sync_copy(acc_i, out_hbm.at[NW + wid])


NTRIP = CPT // 3

_AGG_OUT = jax.ShapeDtypeStruct((2, NPAD, D), F32)
_AGG_SCRATCH = [
    pltpu.VMEM((CH,), I32), pltpu.VMEM((CH,), I32), pltpu.VMEM((CH,), I32),
    pltpu.VMEM((CH,), I32), pltpu.VMEM((CH,), I32), pltpu.VMEM((CH,), I32),
    pltpu.VMEM((CH, D), F32),
    pltpu.VMEM((CH, D), F32),
    pltpu.VMEM((CH, D), F32),
    pltpu.VMEM_SHARED((NPAD, D), F32),
    pltpu.SemaphoreType.DMA, pltpu.SemaphoreType.DMA,
    pltpu.SemaphoreType.DMA, pltpu.SemaphoreType.DMA,
    pltpu.SemaphoreType.DMA, pltpu.SemaphoreType.DMA,
]


def _sc_aggregate_body(h_hbm, src_hbm, dst_hbm, out_hbm,
                       is0, is1, is2, id0, id1, id2, r0, r1, r2, acc,
                       si0, si1, si2, sg0, sg1, sg2):
    c, s, wid = _ids()
    zero = jnp.zeros((16,), F32)

    def zbody(r, carry):
        for j in range(D // 16):
            r0[r, pl.ds(j * 16, 16)] = zero
        return carry

    lax.fori_loop(0, CH, zbody, 0)
    for t in range(ROWS_PER_TILE // CH):
        pltpu.sync_copy(r0, acc.at[pl.ds(s * ROWS_PER_TILE + t * CH, CH), :])
    plsc.subcore_barrier()

    ebase = wid * EPT

    def idx_fetch(chunk, bi_s, bi_d, sem):
        off = ebase + chunk * CH
        pltpu.async_copy(src_hbm.at[pl.ds(off, CH)], bi_s, sem)
        pltpu.async_copy(dst_hbm.at[pl.ds(off, CH)], bi_d, sem)

    def idx_wait(chunk, bi_s, bi_d, sem):
        off = ebase + chunk * CH
        pltpu.make_async_copy(src_hbm.at[pl.ds(off, CH)], bi_s, sem).wait()
        pltpu.make_async_copy(dst_hbm.at[pl.ds(off, CH)], bi_d, sem).wait()

    def gather(bi_s, buf, sem):
        pltpu.async_copy(h_hbm.at[bi_s], buf, sem)

    def gwait(bi_s, buf, sem):
        pltpu.make_async_copy(h_hbm.at[bi_s], buf, sem).wait()

    idx_fetch(0, is0, id0, si0)
    idx_fetch(1, is1, id1, si1)
    idx_fetch(2, is2, id2, si2)
    idx_wait(0, is0, id0, si0)
    gather(is0, r0, sg0)
    idx_wait(1, is1, id1, si1)
    gather(is1, r1, sg1)

    def trip(kk, carry):
        c0 = 3 * kk
        idx_wait(c0 + 2, is2, id2, si2)
        gather(is2, r2, sg2)
        gwait(is0, r0, sg0)
        pltpu.sync_copy(r0, acc.at[id0], add=True)
        idx_fetch(c0 + 3, is0, id0, si0)
        gwait(is1, r1, sg1)
        pltpu.sync_copy(r1, acc.at[id1], add=True)
        idx_fetch(c0 + 4, is1, id1, si1)
        idx_wait(c0 + 3, is0, id0, si0)
        gather(is0, r0, sg0)
        gwait(is2, r2, sg2)
        pltpu.sync_copy(r2, acc.at[id2], add=True)

        @pl.when(kk < NTRIP - 1)
        def _():
            idx_fetch(c0 + 5, is2, id2, si2)

        idx_wait(c0 + 4, is1, id1, si1)
        gather(is1, r1, sg1)
        return carry

    lax.fori_loop(0, NTRIP, trip, 0)

    gwait(is0, r0, sg0)
    pltpu.sync_copy(r0, acc.at[id0], add=True)
    gwait(is1, r1, sg1)
    pltpu.sync_copy(r1, acc.at[id1], add=True)

    plsc.subcore_barrier()
    for t in range(ROWS_PER_TILE // CH):
        off = s * ROWS_PER_TILE + t * CH
        pltpu.sync_copy(acc.at[pl.ds(off, CH), :],
                        out_hbm.at[c, pl.ds(off, CH), :])


_SCORE_OUT = jax.ShapeDtypeStruct((E,), F32)
_SCORE_SCRATCH = [
    pltpu.VMEM((EPT,), I32),
    pltpu.VMEM((EPT,), I32),
    pltpu.VMEM((CH, D), F32), pltpu.VMEM((CH, D), F32),
    pltpu.VMEM((CH, D), F32), pltpu.VMEM((CH, D), F32),
    pltpu.VMEM((CH, D), F32), pltpu.VMEM((CH, D), F32),
    pltpu.VMEM((256,), F32),
    pltpu.VMEM((EPT,), F32),
    pltpu.SemaphoreType.DMA, pltpu.SemaphoreType.DMA,
    pltpu.SemaphoreType.DMA, pltpu.SemaphoreType.DMA,
    pltpu.SemaphoreType.DMA, pltpu.SemaphoreType.DMA,
]


def _sc_scores_body(x_hbm, src_hbm, dst_hbm, out_hbm, idx_s, idx_d,
                    s0, s1, s2, d0, d1, d2, pad, oall,
                    sa0, sa1, sa2, sb0, sb1, sb2):
    c, s, wid = _ids()
    lanes = lax.iota(I32, 16)

    ebase = wid * EPT
    pltpu.sync_copy(src_hbm.at[pl.ds(ebase, EPT)], idx_s)
    pltpu.sync_copy(dst_hbm.at[pl.ds(ebase, EPT)], idx_d)

    def gathers(chunk, bs, bd, sema, semb):
        pltpu.async_copy(x_hbm.at[idx_s.at[pl.ds(chunk * CH, CH)]], bs, sema)
        pltpu.async_copy(x_hbm.at[idx_d.at[pl.ds(chunk * CH, CH)]], bd, semb)

    def gwait(chunk, bs, bd, sema, semb):
        pltpu.make_async_copy(
            x_hbm.at[idx_s.at[pl.ds(chunk * CH, CH)]], bs, sema).wait()
        pltpu.make_async_copy(
            x_hbm.at[idx_d.at[pl.ds(chunk * CH, CH)]], bd, semb).wait()

    def compute(chunk, bs, bd):
        def grp(e16, carry):
            base = e16 * 16
            for e in range(16):
                eidx = base + e
                acc = bs[eidx, pl.ds(0, 16)] * bd[eidx, pl.ds(0, 16)]
                for j in range(1, D // 16):
                    acc = acc + (bs[eidx, pl.ds(j * 16, 16)]
                                 * bd[eidx, pl.ds(j * 16, 16)])
                plsc.store_scatter(pad, [lanes * 16 + e], acc)
            dots = pad[pl.ds(0, 16)]
            for r in range(1, 16):
                dots = dots + pad[pl.ds(r * 16, 16)]
            sig = 1.0 / (1.0 + jnp.exp(-dots))
            oall[pl.ds(chunk * CH + base, 16)] = sig
            return carry

        lax.fori_loop(0, CH // 16, grp, 0)

    gathers(0, s0, d0, sa0, sb0)
    gathers(1, s1, d1, sa1, sb1)

    def trip(kk, carry):
        c0 = 3 * kk
        gathers(c0 + 2, s2, d2, sa2, sb2)
        gwait(c0, s0, d0, sa0, sb0)
        compute(c0, s0, d0)
        gathers(c0 + 3, s0, d0, sa0, sb0)
        gwait(c0 + 1, s1, d1, sa1, sb1)
        compute(c0 + 1, s1, d1)
        gathers(c0 + 4, s1, d1, sa1, sb1)
        gwait(c0 + 2, s2, d2, sa2, sb2)
        compute(c0 + 2, s2, d2)
        return carry

    lax.fori_loop(0, NTRIP, trip, 0)

    last = CPT - 2
    gwait(last, s0, d0, sa0, sb0)
    compute(last, s0, d0)
    gwait(last + 1, s1, d1, sa1, sb1)
    compute(last + 1, s1, d1)

    pltpu.sync_copy(oall, out_hbm.at[pl.ds(ebase, EPT)])


def _make_sc(body, out_type, scratch):
    return pl.kernel(
        body,
        out_type=out_type,
        mesh=_mesh,
        compiler_params=_sc_params,
        scratch_types=scratch,
    )


_sc_degrees = _make_sc(_sc_degrees_body, _DEG_OUT, _DEG_SCRATCH)
_sc_aggregate = _make_sc(_sc_aggregate_body, _AGG_OUT, _AGG_SCRATCH)
_sc_scores = _make_sc(_sc_scores_body, _SCORE_OUT, _SCORE_SCRATCH)


_BLK = 1024
_GRID = NPAD // _BLK


def _scales(degp):
    dego = jnp.sum(degp[0:NW], axis=0)[:, None]
    degi = jnp.sum(degp[NW:2 * NW], axis=0)[:, None]
    s_out = lax.rsqrt(jnp.maximum(dego, 1.0))
    s_in = lax.rsqrt(jnp.maximum(degi, 1.0))
    return s_out, s_in


def _tc_mm_body(feat_ref, w1_ref, out_ref):
    out_ref[...] = jnp.dot(feat_ref[...], w1_ref[...],
                           preferred_element_type=F32)


def _tc_scale_body(degp_ref, mm_ref, out_ref):
    s_out, _ = _scales(degp_ref[...])
    out_ref[...] = mm_ref[...] * s_out


def _tc_mid_body(degp_ref, aggp_ref, b1_ref, w2_ref, out_ref):
    s_out, s_in = _scales(degp_ref[...])
    x = aggp_ref[0] + aggp_ref[1]
    x1 = jnp.maximum(x * s_in + b1_ref[...], 0.0)
    out_ref[...] = jnp.dot(x1 * s_out, w2_ref[...],
                           preferred_element_type=F32)


def _tc_fin_body(degp_ref, aggp_ref, b2_ref, out_ref):
    _, s_in = _scales(degp_ref[...])
    x = aggp_ref[0] + aggp_ref[1]
    out_ref[...] = jnp.maximum(x * s_in + b2_ref[...], 0.0)


_deg_spec = pl.BlockSpec((2 * NW, _BLK), lambda i: (0, i))
_row_spec = pl.BlockSpec((_BLK, D), lambda i: (i, 0))
_agg_spec = pl.BlockSpec((2, _BLK, D), lambda i: (0, i, 0))
_w_spec = pl.BlockSpec((D, D), lambda i: (0, 0))
_b_spec = pl.BlockSpec((1, D), lambda i: (0, 0))

_tc_mm = pl.pallas_call(
    _tc_mm_body,
    grid=(_GRID,),
    in_specs=[_row_spec, _w_spec],
    out_specs=_row_spec,
    out_shape=jax.ShapeDtypeStruct((NPAD, D), F32),
)

_tc_scale = pl.pallas_call(
    _tc_scale_body,
    grid=(_GRID,),
    in_specs=[_deg_spec, _row_spec],
    out_specs=_row_spec,
    out_shape=jax.ShapeDtypeStruct((NPAD, D), F32),
)

_tc_mid = pl.pallas_call(
    _tc_mid_body,
    grid=(_GRID,),
    in_specs=[_deg_spec, _agg_spec, _b_spec, _w_spec],
    out_specs=_row_spec,
    out_shape=jax.ShapeDtypeStruct((NPAD, D), F32),
)

_tc_fin = pl.pallas_call(
    _tc_fin_body,
    grid=(_GRID,),
    in_specs=[_deg_spec, _agg_spec, _b_spec],
    out_specs=_row_spec,
    out_shape=jax.ShapeDtypeStruct((NPAD, D), F32),
)


def kernel(features, edge_index, edge_type, W1, b1, W2, b2):
    del edge_type
    src = edge_index[0]
    dst = edge_index[1]
    featp = jnp.pad(features, ((0, NPAD - N), (0, 0)))

    degp = _sc_degrees(src, dst)
    mm1 = _tc_mm(featp, W1)
    h1 = _tc_scale(degp, mm1)
    agg1 = _sc_aggregate(h1, src, dst)
    h2 = _tc_mid(degp, agg1, b1.reshape(1, D), W2)
    agg2 = _sc_aggregate(h2, src, dst)
    x2 = _tc_fin(degp, agg2, b2.reshape(1, D))
    scores = _sc_scores(x2, src, dst)
    return scores
